# Initial kernel scaffold; baseline (speedup 1.0000x reference)
import jax
import jax.numpy as jnp
from jax.experimental import pallas as pl


def kernel(fe, edge_index, W, b):
    raise NotImplementedError("write your pallas kernel here")


# speedup vs baseline: 2.7106x; 2.7106x over previous
import functools

import jax
import jax.numpy as jnp
from jax import lax
from jax.experimental import pallas as pl
from jax.experimental.pallas import tpu as pltpu
from jax.experimental.pallas import tpu_sc as plsc

C = 128
E = 160000
NUM_WORKERS = 32
GATHER_CHUNK = 400
TC_BLOCK = 3200


def _sc_gather(table, idx_flat):
    n_rows = idx_flat.shape[0]
    per_w = n_rows // NUM_WORKERS
    n_chunks = per_w // GATHER_CHUNK
    mesh = plsc.VectorSubcoreMesh(core_axis_name="c", subcore_axis_name="s")

    @functools.partial(
        pl.kernel,
        mesh=mesh,
        out_type=jax.ShapeDtypeStruct((n_rows, C), jnp.float32),
        scratch_types=[
            pltpu.VMEM((per_w,), jnp.int32),
            pltpu.VMEM((GATHER_CHUNK, C), jnp.float32),
            pltpu.SemaphoreType.DMA,
        ],
    )
    def gather_kernel(table_hbm, idx_hbm, out_hbm, idx_v, rows_v, sem):
        wid = lax.axis_index("s") * 2 + lax.axis_index("c")
        base = wid * per_w
        pltpu.sync_copy(idx_hbm.at[pl.ds(base, per_w)], idx_v)

        def body(i, carry):
            off = i * GATHER_CHUNK
            pltpu.async_copy(
                table_hbm.at[idx_v.at[pl.ds(off, GATHER_CHUNK)]], rows_v, sem
            ).wait()
            pltpu.sync_copy(rows_v, out_hbm.at[pl.ds(base + off, GATHER_CHUNK)])
            return carry

        lax.fori_loop(0, n_chunks, body, 0)

    return gather_kernel(table, idx_flat)


def _conv_pass(fe2, nb, wstack):
    n_blocks = E // TC_BLOCK

    def body(fe_ref, nb_ref, w_ref, y_ref, s1_ref, s2_ref):
        i = pl.program_id(0)
        x0 = fe_ref[...]
        a0 = nb_ref[0]
        a1 = nb_ref[1]
        a2 = nb_ref[2]
        a3 = nb_ref[3]
        f1 = a0 + a2
        f2 = a1 + a3
        f3 = jnp.abs(a0 - a2)
        f4 = jnp.abs(a1 - a3)
        dn = (((1,), (1,)), ((), ()))
        y = jax.lax.dot(w_ref[0], x0, precision=jax.lax.Precision.HIGHEST)
        for w_idx, f in ((1, f1), (2, f2), (3, f3), (4, f4)):
            y = y + jax.lax.dot_general(
                w_ref[w_idx], f, dn, precision=jax.lax.Precision.HIGHEST
            )
        y_ref[...] = y

        @pl.when(i == 0)
        def _():
            s1_ref[...] = jnp.zeros_like(s1_ref)
            s2_ref[...] = jnp.zeros_like(s2_ref)

        s1_ref[...] += jnp.sum(y, axis=1, keepdims=True)
        s2_ref[...] += jnp.sum(y * y, axis=1, keepdims=True)

    return pl.pallas_call(
        body,
        grid=(n_blocks,),
        in_specs=[
            pl.BlockSpec((C, TC_BLOCK), lambda i: (0, i)),
            pl.BlockSpec((4, TC_BLOCK, C), lambda i: (0, i, 0)),
            pl.BlockSpec((5, C, C), lambda i: (0, 0, 0)),
        ],
        out_specs=[
            pl.BlockSpec((C, TC_BLOCK), lambda i: (0, i)),
            pl.BlockSpec((C, 1), lambda i: (0, 0)),
            pl.BlockSpec((C, 1), lambda i: (0, 0)),
        ],
        out_shape=[
            jax.ShapeDtypeStruct((C, E), jnp.float32),
            jax.ShapeDtypeStruct((C, 1), jnp.float32),
            jax.ShapeDtypeStruct((C, 1), jnp.float32),
        ],
    )(fe2, nb, wstack)


def _norm_pass(y, s1, s2):
    n_blocks = E // TC_BLOCK
    inv_e = 1.0 / E

    def body(y_ref, s1_ref, s2_ref, o_ref):
        mean = s1_ref[...] * inv_e
        var = s2_ref[...] * inv_e - mean * mean
        inv = lax.rsqrt(var + 1e-5)
        o_ref[...] = jnp.maximum((y_ref[...] - mean) * inv, 0.0)

    return pl.pallas_call(
        body,
        grid=(n_blocks,),
        in_specs=[
            pl.BlockSpec((C, TC_BLOCK), lambda i: (0, i)),
            pl.BlockSpec((C, 1), lambda i: (0, 0)),
            pl.BlockSpec((C, 1), lambda i: (0, 0)),
        ],
        out_specs=pl.BlockSpec((C, TC_BLOCK), lambda i: (0, i)),
        out_shape=jax.ShapeDtypeStruct((C, E), jnp.float32),
    )(y, s1, s2)


def kernel(fe, edge_index, W, b):
    del b
    table = fe[0].T
    idx_flat = edge_index[0].T.reshape(-1)
    nb_flat = _sc_gather(table, idx_flat)
    nb = nb_flat.reshape(4, E, C)
    wstack = jnp.moveaxis(W[:, :, 0, :], -1, 0)
    y, s1, s2 = _conv_pass(fe[0], nb, wstack)
    out = _norm_pass(y, s1, s2)
    return out[None]
